# baseline (speedup 1.0000x reference)
import functools

import jax
import jax.numpy as jnp
from jax import lax
from jax.experimental import pallas as pl
from jax.experimental.pallas import tpu as pltpu
from jax.experimental.pallas import tpu_sc as plsc

_BKL = 30976
_HSUB = 4
_NBUF = 3
_SC_F = 32768


def _scores(w1, w2, ones, ct):
    ct_hi = ct.astype(jnp.bfloat16).astype(jnp.float32)
    ct_lo = ct - ct_hi
    sq = ct * ct
    sq_hi = sq.astype(jnp.bfloat16).astype(jnp.float32)
    sq_lo = sq - sq_hi

    def mm(a, b):
        return jax.lax.dot_general(
            a, b, (((1,), (0,)), ((), ())),
            preferred_element_type=jnp.float32)

    return (mm(w1, ct_hi) + mm(w1, ct_lo)) + (
        mm(w2, ct_hi) + (mm(ones, sq_hi) + mm(ones, sq_lo)))


def _argmin_lanes(scores, base, num_rows):
    local_min = jnp.min(scores, axis=1, keepdims=True)
    lane_ids = jax.lax.broadcasted_iota(jnp.int32, scores.shape, 1)
    masked = jnp.where(scores == local_min, lane_ids, num_rows)
    local_arg = jnp.min(masked, axis=1, keepdims=True)
    return local_min, base + local_arg


def _nn_kernel(w1_ref, w2_ref, tail_ref, ct_hbm, val_ref, idx_ref, bufs,
               sems, val_s, idx_s, *, tc_base, bkl, nbuf, nstep, num_rows,
               tail, tail_base):
    i = pl.program_id(0)
    w1 = w1_ref[...]
    w2 = w2_ref[...]
    ones = jnp.ones_like(w1)

    @pl.when(i == 0)
    def _prologue():
        for b in range(min(nbuf, nstep)):
            pltpu.make_async_copy(
                ct_hbm.at[:, pl.ds(tc_base + b * bkl, bkl)], bufs.at[b], sems.at[b]
            ).start()
        if tail:
            tv, ti = _argmin_lanes(
                _scores(w1, w2, ones, tail_ref[...]), tail_base, num_rows)
            val_s[...] = tv
            idx_s[...] = ti
        else:
            val_s[...] = jnp.full_like(val_s, jnp.inf)
            idx_s[...] = jnp.zeros_like(idx_s)

    slot = jax.lax.rem(i, nbuf)
    pltpu.make_async_copy(
        ct_hbm.at[:, pl.ds(tc_base + i * bkl, bkl)], bufs.at[slot], sems.at[slot]
    ).wait()

    hsub = _HSUB
    sub = bkl // hsub
    cbuf = bufs[slot]
    for h in range(hsub):
        local_min, local_idx = _argmin_lanes(
            _scores(w1, w2, ones, cbuf[:, h * sub:(h + 1) * sub]),
            tc_base + i * bkl + h * sub, num_rows)

        prev_v = val_s[...]
        prev_i = idx_s[...]
        better = (local_min < prev_v) | (
            (local_min == prev_v) & (local_idx < prev_i))
        val_s[...] = jnp.where(better, local_min, prev_v)
        idx_s[...] = jnp.where(better, local_idx, prev_i)

    @pl.when(i + nbuf < nstep)
    def _refill():
        pltpu.make_async_copy(
            ct_hbm.at[:, pl.ds(tc_base + (i + nbuf) * bkl, bkl)], bufs.at[slot],
            sems.at[slot]
        ).start()

    @pl.when(i == nstep - 1)
    def _finish():
        val_ref[...] = val_s[...]
        idx_ref[...] = idx_s[...]


def _tc_call(w1, w2, ct, tc_base, num_rows):
    p, d = w1.shape
    bkl = _BKL
    nbuf = _NBUF
    span = num_rows - tc_base
    nstep = span // bkl
    main = nstep * bkl
    tail = span - main
    tail_base = tc_base + main
    tail_arr = (ct[:, tail_base:num_rows] if tail
                else jnp.zeros((d, 1), jnp.float32))
    tail_w = tail if tail else 1

    return pl.pallas_call(
        functools.partial(_nn_kernel, tc_base=tc_base, bkl=bkl, nbuf=nbuf,
                          nstep=nstep, num_rows=num_rows, tail=tail,
                          tail_base=tail_base),
        grid=(nstep,),
        in_specs=[
            pl.BlockSpec((p, d), lambda i: (0, 0)),
            pl.BlockSpec((p, d), lambda i: (0, 0)),
            pl.BlockSpec((d, tail_w), lambda i: (0, 0)),
            pl.BlockSpec(memory_space=pl.ANY),
        ],
        out_specs=[
            pl.BlockSpec((p, 1), lambda i: (0, 0)),
            pl.BlockSpec((p, 1), lambda i: (0, 0)),
        ],
        out_shape=[
            jax.ShapeDtypeStruct((p, 1), jnp.float32),
            jax.ShapeDtypeStruct((p, 1), jnp.int32),
        ],
        scratch_shapes=[
            pltpu.VMEM((nbuf, d, bkl), jnp.float32),
            pltpu.SemaphoreType.DMA((nbuf,)),
            pltpu.VMEM((p, 1), jnp.float32),
            pltpu.VMEM((p, 1), jnp.int32),
        ],
    )(w1, w2, tail_arr, ct)


def _sc_nn(m2p, ct, base, w, num_rows):
    d, p = m2p.shape
    groups = w // 16
    mesh = plsc.VectorSubcoreMesh(core_axis_name="c", subcore_axis_name="s")

    @functools.partial(
        pl.kernel, mesh=mesh,
        out_type=[
            jax.ShapeDtypeStruct((32, p, 16), jnp.float32),
            jax.ShapeDtypeStruct((32, p, 16), jnp.int32),
        ],
        scratch_types=[
            pltpu.VMEM((d, w), jnp.float32),
            pltpu.VMEM((d, p), jnp.float32),
            pltpu.VMEM((p, 16), jnp.float32),
            pltpu.VMEM((p, 16), jnp.int32),
            pltpu.SemaphoreType.DMA,
        ],
    )
    def sc_k(ct_hbm, m2pt_hbm, val_hbm, idx_hbm, buf, pvm, stv, sti, sem):
        cix = lax.axis_index("c")
        six = lax.axis_index("s")
        wid = six * 2 + cix
        row0 = base + wid * w
        pltpu.sync_copy(m2pt_hbm, pvm)
        pltpu.async_copy(ct_hbm.at[:, pl.ds(row0, w)], buf, sem).wait()

        lane = lax.iota(jnp.int32, 16)
        big = jnp.float32(jnp.inf)

        def group_body(g, carry):
            mins = carry[:16]
            idxs = carry[16:]
            gbase = g * 16

            def d_body(dd, dc):
                accs = dc[:16]
                c2 = dc[16]
                v = buf[dd, pl.ds(gbase, 16)]
                pv = pvm[dd, pl.ds(0, 16)]
                new = tuple(accs[j] + v * pv[j] for j in range(16))
                return new + (c2 + v * v,)

            zero = jnp.zeros((16,), jnp.float32)
            dc = lax.fori_loop(0, d, d_body, tuple(zero for _ in range(16))
                               + (zero,))
            c2 = dc[16]
            idvec = lane + (row0 + gbase)
            nmins = []
            nidxs = []
            for j in range(16):
                s_j = c2 + dc[j]
                m = s_j < mins[j]
                nmins.append(jnp.where(m, s_j, mins[j]))
                nidxs.append(jnp.where(m, idvec, idxs[j]))
            return tuple(nmins) + tuple(nidxs)

        inf = jnp.full((16,), big, jnp.float32)
        zi = jnp.zeros((16,), jnp.int32)
        carry = lax.fori_loop(
            0, groups, group_body,
            tuple(inf for _ in range(16)) + tuple(zi for _ in range(16)))

        for j in range(16):
            stv[j, pl.ds(0, 16)] = carry[j]
            sti[j, pl.ds(0, 16)] = carry[16 + j]

        pltpu.sync_copy(stv, val_hbm.at[wid])
        pltpu.sync_copy(sti, idx_hbm.at[wid])

    return sc_k(ct, m2p)


def kernel(prompt_embs, clip_embs):
    num_rows, d = clip_embs.shape
    p = prompt_embs.shape[0]
    ct = clip_embs.T

    sc_f = _SC_F
    sc_w = sc_f // 32

    p_hi = prompt_embs.astype(jnp.bfloat16).astype(jnp.float32)
    p_lo = (prompt_embs - p_hi).astype(jnp.bfloat16).astype(jnp.float32)
    w1 = -2.0 * p_hi
    w2 = -2.0 * p_lo
    m2p = -2.0 * prompt_embs

    sc_val, sc_idx = _sc_nn(m2p.T, ct, 0, sc_w, num_rows)
    tc_val, tc_idx = _tc_call(w1, w2, ct, sc_f, num_rows)

    scv = jnp.transpose(sc_val, (0, 2, 1)).reshape(-1, p)
    sci = jnp.transpose(sc_idx, (0, 2, 1)).reshape(-1, p)
    allv = jnp.concatenate([tc_val.T, scv], axis=0)
    alli = jnp.concatenate([tc_idx.T, sci], axis=0)
    m = jnp.min(allv, axis=0)
    ids = jnp.min(jnp.where(allv == m, alli, num_rows), axis=0)
    return (prompt_embs, prompt_embs, ids.astype(jnp.int32))


# speedup vs baseline: 1.6401x; 1.6401x over previous
import functools

import jax
import jax.numpy as jnp
from jax.experimental import pallas as pl
from jax.experimental.pallas import tpu as pltpu

_BKL = 62464
_HSUB = 8
_NBUF = 2


def _scores(w1, w2, ones, ct):
    ct_hi = ct.astype(jnp.bfloat16).astype(jnp.float32)
    ct_lo = ct - ct_hi
    sq = ct * ct
    sq_hi = sq.astype(jnp.bfloat16).astype(jnp.float32)
    sq_lo = sq - sq_hi

    def mm(a, b):
        return jax.lax.dot_general(
            a, b, (((1,), (0,)), ((), ())),
            preferred_element_type=jnp.float32)

    return (mm(w1, ct_hi) + mm(w1, ct_lo)) + (
        mm(w2, ct_hi) + (mm(ones, sq_hi) + mm(ones, sq_lo)))


def _argmin_lanes(scores, base, num_rows):
    local_min = jnp.min(scores, axis=1, keepdims=True)
    lane_ids = jax.lax.broadcasted_iota(jnp.int32, scores.shape, 1)
    masked = jnp.where(scores == local_min, lane_ids, num_rows)
    local_arg = jnp.min(masked, axis=1, keepdims=True)
    return local_min, base + local_arg


def _nn_kernel(w1_ref, w2_ref, tail_ref, ct_hbm, idx_ref, bufs, sems,
               val_s, idx_s, *, bkl, nbuf, nstep, num_rows, tail):
    i = pl.program_id(0)
    w1 = w1_ref[...]
    w2 = w2_ref[...]
    ones = jnp.ones_like(w1)

    @pl.when(i == 0)
    def _prologue():
        for b in range(min(nbuf, nstep)):
            pltpu.make_async_copy(
                ct_hbm.at[:, pl.ds(b * bkl, bkl)], bufs.at[b], sems.at[b]
            ).start()
        if tail:
            tv, ti = _argmin_lanes(
                _scores(w1, w2, ones, tail_ref[...]),
                nstep * bkl, num_rows)
            val_s[...] = tv
            idx_s[...] = ti
        else:
            val_s[...] = jnp.full_like(val_s, jnp.inf)
            idx_s[...] = jnp.zeros_like(idx_s)

    slot = jax.lax.rem(i, nbuf)
    pltpu.make_async_copy(
        ct_hbm.at[:, pl.ds(i * bkl, bkl)], bufs.at[slot], sems.at[slot]
    ).wait()

    hsub = _HSUB
    sub = bkl // hsub
    cbuf = bufs[slot]
    for h in range(hsub):
        local_min, local_idx = _argmin_lanes(
            _scores(w1, w2, ones, cbuf[:, h * sub:(h + 1) * sub]),
            i * bkl + h * sub, num_rows)

        prev_v = val_s[...]
        prev_i = idx_s[...]
        better = (local_min < prev_v) | (
            (local_min == prev_v) & (local_idx < prev_i))
        val_s[...] = jnp.where(better, local_min, prev_v)
        idx_s[...] = jnp.where(better, local_idx, prev_i)

    @pl.when(i + nbuf < nstep)
    def _refill():
        pltpu.make_async_copy(
            ct_hbm.at[:, pl.ds((i + nbuf) * bkl, bkl)], bufs.at[slot],
            sems.at[slot]
        ).start()

    @pl.when(i == nstep - 1)
    def _finish():
        idx_ref[...] = idx_s[...]


def kernel(prompt_embs, clip_embs):
    num_rows, d = clip_embs.shape
    p = prompt_embs.shape[0]
    bkl = _BKL
    nbuf = _NBUF
    nstep = num_rows // bkl
    main = nstep * bkl
    tail = num_rows - main
    ct = clip_embs.T
    tail_arr = ct[:, main:] if tail else jnp.zeros((d, 1), jnp.float32)
    tail_w = tail if tail else 1

    p_hi = prompt_embs.astype(jnp.bfloat16).astype(jnp.float32)
    p_lo = (prompt_embs - p_hi).astype(jnp.bfloat16).astype(jnp.float32)
    w1 = -2.0 * p_hi
    w2 = -2.0 * p_lo

    idx = pl.pallas_call(
        functools.partial(_nn_kernel, bkl=bkl, nbuf=nbuf, nstep=nstep,
                          num_rows=num_rows, tail=tail),
        grid=(nstep,),
        in_specs=[
            pl.BlockSpec((p, d), lambda i: (0, 0)),
            pl.BlockSpec((p, d), lambda i: (0, 0)),
            pl.BlockSpec((d, tail_w), lambda i: (0, 0)),
            pl.BlockSpec(memory_space=pl.ANY),
        ],
        out_specs=pl.BlockSpec((p, 1), lambda i: (0, 0)),
        out_shape=jax.ShapeDtypeStruct((p, 1), jnp.int32),
        scratch_shapes=[
            pltpu.VMEM((nbuf, d, bkl), jnp.float32),
            pltpu.SemaphoreType.DMA((nbuf,)),
            pltpu.VMEM((p, 1), jnp.float32),
            pltpu.VMEM((p, 1), jnp.int32),
        ],
    )(w1, w2, tail_arr, ct)

    ids = idx[:, 0]
    return (prompt_embs, prompt_embs, ids)


# speedup vs baseline: 1.6432x; 1.0019x over previous
import functools

import jax
import jax.numpy as jnp
from jax.experimental import pallas as pl
from jax.experimental.pallas import tpu as pltpu

_BKL = 62464
_HSUB = 8
_NBUF = 2


def _scores(w1, w2, ones, ct):
    ct_hi = ct.astype(jnp.bfloat16).astype(jnp.float32)
    ct_lo = ct - ct_hi
    sq = ct * ct
    sq_hi = sq.astype(jnp.bfloat16).astype(jnp.float32)
    sq_lo = sq - sq_hi

    def mm(a, b):
        return jax.lax.dot_general(
            a, b, (((1,), (0,)), ((), ())),
            preferred_element_type=jnp.float32)

    return (mm(w1, ct_hi) + mm(w1, ct_lo)) + (
        mm(w2, ct_hi) + (mm(ones, sq_hi) + mm(ones, sq_lo)))


def _argmin_lanes(scores, base, num_rows):
    local_min = jnp.min(scores, axis=1, keepdims=True)
    lane_ids = jax.lax.broadcasted_iota(jnp.int32, scores.shape, 1)
    masked = jnp.where(scores == local_min, lane_ids, num_rows)
    local_arg = jnp.min(masked, axis=1, keepdims=True)
    return local_min, base + local_arg


def _nn_kernel(w1_ref, w2_ref, tail_ref, ct_hbm, idx_ref, bufs, sems,
               val_s, idx_s, *, bkl, nbuf, nstep, num_rows, tail):
    i = pl.program_id(0)
    w1 = w1_ref[...]
    w2 = w2_ref[...]
    ones = jnp.ones_like(w1)

    half = bkl // 2

    def _start(step, b):
        pltpu.make_async_copy(
            ct_hbm.at[:, pl.ds(step * bkl, half)],
            bufs.at[b, :, pl.ds(0, half)], sems.at[b, 0]).start()
        pltpu.make_async_copy(
            ct_hbm.at[:, pl.ds(step * bkl + half, half)],
            bufs.at[b, :, pl.ds(half, half)], sems.at[b, 1]).start()

    @pl.when(i == 0)
    def _prologue():
        for b in range(min(nbuf, nstep)):
            _start(b, b)
        if tail:
            tv, ti = _argmin_lanes(
                _scores(w1, w2, ones, tail_ref[...]),
                nstep * bkl, num_rows)
            val_s[...] = tv
            idx_s[...] = ti
        else:
            val_s[...] = jnp.full_like(val_s, jnp.inf)
            idx_s[...] = jnp.zeros_like(idx_s)

    slot = jax.lax.rem(i, nbuf)
    pltpu.make_async_copy(
        ct_hbm.at[:, pl.ds(i * bkl, half)],
        bufs.at[slot, :, pl.ds(0, half)], sems.at[slot, 0]).wait()
    pltpu.make_async_copy(
        ct_hbm.at[:, pl.ds(i * bkl + half, half)],
        bufs.at[slot, :, pl.ds(half, half)], sems.at[slot, 1]).wait()

    hsub = _HSUB
    sub = bkl // hsub
    cbuf = bufs[slot]
    for h in range(hsub):
        local_min, local_idx = _argmin_lanes(
            _scores(w1, w2, ones, cbuf[:, h * sub:(h + 1) * sub]),
            i * bkl + h * sub, num_rows)

        prev_v = val_s[...]
        prev_i = idx_s[...]
        better = (local_min < prev_v) | (
            (local_min == prev_v) & (local_idx < prev_i))
        val_s[...] = jnp.where(better, local_min, prev_v)
        idx_s[...] = jnp.where(better, local_idx, prev_i)

    @pl.when(i + nbuf < nstep)
    def _refill():
        _start(i + nbuf, slot)

    @pl.when(i == nstep - 1)
    def _finish():
        idx_ref[...] = idx_s[...]


def kernel(prompt_embs, clip_embs):
    num_rows, d = clip_embs.shape
    p = prompt_embs.shape[0]
    bkl = _BKL
    nbuf = _NBUF
    nstep = num_rows // bkl
    main = nstep * bkl
    tail = num_rows - main
    ct = clip_embs.T
    tail_arr = ct[:, main:] if tail else jnp.zeros((d, 1), jnp.float32)
    tail_w = tail if tail else 1

    p_hi = prompt_embs.astype(jnp.bfloat16).astype(jnp.float32)
    p_lo = (prompt_embs - p_hi).astype(jnp.bfloat16).astype(jnp.float32)
    w1 = -2.0 * p_hi
    w2 = -2.0 * p_lo

    idx = pl.pallas_call(
        functools.partial(_nn_kernel, bkl=bkl, nbuf=nbuf, nstep=nstep,
                          num_rows=num_rows, tail=tail),
        grid=(nstep,),
        in_specs=[
            pl.BlockSpec((p, d), lambda i: (0, 0)),
            pl.BlockSpec((p, d), lambda i: (0, 0)),
            pl.BlockSpec((d, tail_w), lambda i: (0, 0)),
            pl.BlockSpec(memory_space=pl.ANY),
        ],
        out_specs=pl.BlockSpec((p, 1), lambda i: (0, 0)),
        out_shape=jax.ShapeDtypeStruct((p, 1), jnp.int32),
        scratch_shapes=[
            pltpu.VMEM((nbuf, d, bkl), jnp.float32),
            pltpu.SemaphoreType.DMA((nbuf, 2)),
            pltpu.VMEM((p, 1), jnp.float32),
            pltpu.VMEM((p, 1), jnp.int32),
        ],
    )(w1, w2, tail_arr, ct)

    ids = idx[:, 0]
    return (prompt_embs, prompt_embs, ids)


# speedup vs baseline: 1.6733x; 1.0183x over previous
import functools

import jax
import jax.numpy as jnp
from jax.experimental import pallas as pl
from jax.experimental.pallas import tpu as pltpu

_BKL = 62464
_HSUB = 8
_NBUF = 2


def _scores(w1, w2, ones, ct):
    ct_hi = ct.astype(jnp.bfloat16).astype(jnp.float32)
    ct_lo = ct - ct_hi
    sq = ct * ct
    sq_hi = sq.astype(jnp.bfloat16).astype(jnp.float32)
    sq_lo = sq - sq_hi

    def mm(a, b):
        return jax.lax.dot_general(
            a, b, (((1,), (0,)), ((), ())),
            preferred_element_type=jnp.float32)

    return (mm(w1, ct_hi) + mm(w1, ct_lo)) + (
        mm(w2, ct_hi) + (mm(ones, sq_hi) + mm(ones, sq_lo)))


def _argmin_lanes(scores, base, num_rows):
    local_min = jnp.min(scores, axis=1, keepdims=True)
    lane_ids = jax.lax.broadcasted_iota(jnp.int32, scores.shape, 1)
    masked = jnp.where(scores == local_min, lane_ids, num_rows)
    local_arg = jnp.min(masked, axis=1, keepdims=True)
    return local_min, base + local_arg


def _nn_kernel(w1_ref, w2_ref, tail_ref, ct_hbm, idx_ref, bufs, sems,
               val_s, idx_s, *, bkl, nbuf, nstep, num_rows, tail):
    i = pl.program_id(0)
    w1 = w1_ref[...]
    w2 = w2_ref[...]
    ones = jnp.ones_like(w1)

    @pl.when(i == 0)
    def _prologue():
        for b in range(min(nbuf, nstep)):
            pltpu.make_async_copy(
                ct_hbm.at[:, pl.ds(b * bkl, bkl)], bufs.at[b], sems.at[b]
            ).start()
        if tail:
            tv, ti = _argmin_lanes(
                _scores(w1, w2, ones, tail_ref[...]),
                nstep * bkl, num_rows)
            val_s[...] = tv
            idx_s[...] = ti
        else:
            val_s[...] = jnp.full_like(val_s, jnp.inf)
            idx_s[...] = jnp.zeros_like(idx_s)

    slot = jax.lax.rem(i, nbuf)
    pltpu.make_async_copy(
        ct_hbm.at[:, pl.ds(i * bkl, bkl)], bufs.at[slot], sems.at[slot]
    ).wait()

    hsub = _HSUB
    sub = bkl // hsub
    cbuf = bufs[slot]
    for h in range(hsub):
        local_min, local_idx = _argmin_lanes(
            _scores(w1, w2, ones, cbuf[:, h * sub:(h + 1) * sub]),
            i * bkl + h * sub, num_rows)

        prev_v = val_s[...]
        prev_i = idx_s[...]
        better = (local_min < prev_v) | (
            (local_min == prev_v) & (local_idx < prev_i))
        val_s[...] = jnp.where(better, local_min, prev_v)
        idx_s[...] = jnp.where(better, local_idx, prev_i)

    @pl.when(i + nbuf < nstep)
    def _refill():
        pltpu.make_async_copy(
            ct_hbm.at[:, pl.ds((i + nbuf) * bkl, bkl)], bufs.at[slot],
            sems.at[slot]
        ).start()

    @pl.when(i == nstep - 1)
    def _finish():
        idx_ref[...] = idx_s[...]


def kernel(prompt_embs, clip_embs):
    num_rows, d = clip_embs.shape
    p = prompt_embs.shape[0]
    bkl = _BKL
    nbuf = _NBUF
    nstep = num_rows // bkl
    main = nstep * bkl
    tail = num_rows - main
    ct = clip_embs.T
    tail_arr = ct[:, main:] if tail else jnp.zeros((d, 1), jnp.float32)
    tail_w = tail if tail else 1

    p_hi = prompt_embs.astype(jnp.bfloat16).astype(jnp.float32)
    p_lo = (prompt_embs - p_hi).astype(jnp.bfloat16).astype(jnp.float32)
    w1 = -2.0 * p_hi
    w2 = -2.0 * p_lo

    idx = pl.pallas_call(
        functools.partial(_nn_kernel, bkl=bkl, nbuf=nbuf, nstep=nstep,
                          num_rows=num_rows, tail=tail),
        grid=(nstep,),
        in_specs=[
            pl.BlockSpec((p, d), lambda i: (0, 0)),
            pl.BlockSpec((p, d), lambda i: (0, 0)),
            pl.BlockSpec((d, tail_w), lambda i: (0, 0)),
            pl.BlockSpec(memory_space=pl.ANY),
        ],
        out_specs=pl.BlockSpec((p, 1), lambda i: (0, 0)),
        out_shape=jax.ShapeDtypeStruct((p, 1), jnp.int32),
        scratch_shapes=[
            pltpu.VMEM((nbuf, d, bkl), jnp.float32),
            pltpu.SemaphoreType.DMA((nbuf,)),
            pltpu.VMEM((p, 1), jnp.float32),
            pltpu.VMEM((p, 1), jnp.int32),
        ],
    )(w1, w2, tail_arr, ct)

    ids = idx[:, 0]
    return (prompt_embs, prompt_embs, ids)
